# baseline (speedup 1.0000x reference)
import functools

import jax
import jax.numpy as jnp
from jax import lax
from jax.experimental import pallas as pl
from jax.experimental.pallas import tpu as pltpu
from jax.experimental.pallas import tpu_sc as plsc

D = 64
L = 50
_info = plsc.get_sparse_core_info()
NC, NS = _info.num_cores, _info.num_subcores
NW = NC * NS

NB = 8
NBUF = 2


def _make_sc_call(b_total: int):
    rows_w = b_total // NW
    n_chunks = rows_w // NB
    mesh = plsc.VectorSubcoreMesh(core_axis_name="c", subcore_axis_name="s")

    @functools.partial(
        pl.kernel,
        out_type=(
            jax.ShapeDtypeStruct((b_total, L, D), jnp.float32),
            jax.ShapeDtypeStruct((b_total, L, D), jnp.float32),
        ),
        mesh=mesh,
        scratch_types=[
            pltpu.VMEM((NBUF, NB, L), jnp.int32),
            pltpu.VMEM((NBUF, NB, L), jnp.int32),
            pltpu.VMEM((NBUF, NB, L, D), jnp.float32),
            pltpu.VMEM((NBUF, NB, L, D), jnp.float32),
            pltpu.SemaphoreType.DMA((NBUF,)),
            pltpu.SemaphoreType.DMA((NBUF,)),
            pltpu.SemaphoreType.DMA((NBUF,)),
            pltpu.SemaphoreType.DMA((NBUF,)),
        ],
        compiler_params=pltpu.CompilerParams(use_tc_tiling_on_sc=False),
    )
    def sc_call(users_h, types_h, utab_h, ttab_h, out1_h, out2_h,
                idx_u, idx_t, urows, trows, gsem_u, gsem_t, osem1, osem2):
        wid = lax.axis_index("s") * NC + lax.axis_index("c")
        base_w = wid * rows_w

        def fire(ci, b):
            b0 = base_w + ci * NB
            pltpu.sync_copy(users_h.at[pl.ds(b0, NB)], idx_u.at[b])
            pltpu.sync_copy(types_h.at[pl.ds(b0, NB)], idx_t.at[b])
            for r in range(NB):
                pltpu.async_copy(utab_h.at[idx_u.at[b, r]],
                                 urows.at[b, r], gsem_u.at[b])
                pltpu.async_copy(ttab_h.at[idx_t.at[b, r]],
                                 trows.at[b, r], gsem_t.at[b])

        def wait_gathers(b):
            for r in range(NB):
                pltpu.make_async_copy(utab_h.at[idx_u.at[b, r]],
                                      urows.at[b, r], gsem_u.at[b]).wait()
                pltpu.make_async_copy(ttab_h.at[idx_t.at[b, r]],
                                      trows.at[b, r], gsem_t.at[b]).wait()

        def wait_out(ci, b):
            b0 = base_w + ci * NB
            pltpu.make_async_copy(urows.at[b], out1_h.at[pl.ds(b0, NB)],
                                  osem1.at[b]).wait()
            pltpu.make_async_copy(trows.at[b], out2_h.at[pl.ds(b0, NB)],
                                  osem2.at[b]).wait()

        fire(0, 0)

        def superstep(s, carry):
            for bb in range(NBUF):
                i = s * NBUF + bb
                b0 = base_w + i * NB
                wait_gathers(bb)

                nb_slot = (bb + 1) % NBUF

                @pl.when(i + 1 < n_chunks)
                def _(i=i, nb_slot=nb_slot):
                    @pl.when(i >= NBUF - 1)
                    def _():
                        wait_out(i + 1 - NBUF, nb_slot)
                    fire(i + 1, nb_slot)

                def add_row(j, c2, bb=bb):
                    r = j // L
                    l = j - r * L
                    for c in range(D // 16):
                        sl = pl.ds(c * 16, 16)
                        urows[bb, r, l, sl] = (urows[bb, r, l, sl]
                                               + trows[bb, r, l, sl])
                    return c2

                lax.fori_loop(0, NB * L, add_row, 0, unroll=2)
                pltpu.async_copy(urows.at[bb], out1_h.at[pl.ds(b0, NB)],
                                 osem1.at[bb])
                pltpu.async_copy(trows.at[bb], out2_h.at[pl.ds(b0, NB)],
                                 osem2.at[bb])
            return carry

        lax.fori_loop(0, n_chunks // NBUF, superstep, 0)

        for k in range(NBUF):
            ci = n_chunks - NBUF + k
            wait_out(ci, ci % NBUF)

    return sc_call


_sc_call = _make_sc_call(4096)


def _impl(users, items, types, user_table, item_table, type_table):
    del items, item_table
    u = users.astype(jnp.int32)
    t = types.astype(jnp.int32)
    return _sc_call(u, t, user_table, type_table)


_impl.__name__ = "kernel"
_jitted = None


def kernel(users, items, types, user_table, item_table, type_table):
    global _jitted
    if _jitted is None:
        from jax.experimental import layout as jlayout

        sh = jax.sharding.SingleDeviceSharding(jax.devices()[0])
        fmt = jlayout.Format(
            jlayout.Layout(major_to_minor=(0, 1, 2), tiling=((8,),)), sh)
        _jitted = jax.jit(_impl, out_shardings=(fmt, fmt))
    return _jitted(users, items, types, user_table, item_table, type_table)


# speedup vs baseline: 1.0270x; 1.0270x over previous
import functools

import jax
import jax.numpy as jnp
from jax import lax
from jax.experimental import pallas as pl
from jax.experimental.pallas import tpu as pltpu
from jax.experimental.pallas import tpu_sc as plsc

D = 64
DP = 128
L = 50
_info = plsc.get_sparse_core_info()
NC, NS = _info.num_cores, _info.num_subcores
NW = NC * NS

NB = 4
NBUF = 2


def _make_sc_call(b_total: int, v_pad: int):
    rows_w = b_total // NW
    n_chunks = rows_w // NB
    mesh = plsc.VectorSubcoreMesh(core_axis_name="c", subcore_axis_name="s")

    @functools.partial(
        pl.kernel,
        out_type=(
            jax.ShapeDtypeStruct((b_total, L, D), jnp.float32),
            jax.ShapeDtypeStruct((b_total, L, D), jnp.float32),
        ),
        mesh=mesh,
        scratch_types=[
            pltpu.VMEM((NBUF, NB, L), jnp.int32),
            pltpu.VMEM((NBUF, NB, L), jnp.int32),
            pltpu.VMEM((NBUF, NB, L, DP), jnp.float32),
            pltpu.VMEM((NBUF, NB, L, D), jnp.float32),
            pltpu.VMEM((NBUF, NB, L, D), jnp.float32),
            pltpu.SemaphoreType.DMA((NBUF,)),
            pltpu.SemaphoreType.DMA((NBUF,)),
            pltpu.SemaphoreType.DMA((NBUF,)),
            pltpu.SemaphoreType.DMA((NBUF,)),
        ],
        compiler_params=pltpu.CompilerParams(use_tc_tiling_on_sc=False),
    )
    def sc_call(users_h, types_h, utab_h, ttab_h, out1_h, out2_h,
                idx_u, idx_t, upad, trows, usum,
                gsem_u, gsem_t, osem1, osem2):
        wid = lax.axis_index("s") * NC + lax.axis_index("c")
        base_w = wid * rows_w

        def fire(ci, b):
            b0 = base_w + ci * NB
            pltpu.sync_copy(users_h.at[pl.ds(b0, NB)], idx_u.at[b])
            pltpu.sync_copy(types_h.at[pl.ds(b0, NB)], idx_t.at[b])
            for r in range(NB):
                pltpu.async_copy(utab_h.at[idx_u.at[b, r]],
                                 upad.at[b, r], gsem_u.at[b])
                pltpu.async_copy(ttab_h.at[idx_t.at[b, r]],
                                 trows.at[b, r], gsem_t.at[b])

        def wait_gathers(b):
            for r in range(NB):
                pltpu.make_async_copy(utab_h.at[idx_u.at[b, r]],
                                      upad.at[b, r], gsem_u.at[b]).wait()
                pltpu.make_async_copy(ttab_h.at[idx_t.at[b, r]],
                                      trows.at[b, r], gsem_t.at[b]).wait()

        def wait_out(ci, b):
            b0 = base_w + ci * NB
            pltpu.make_async_copy(usum.at[b], out1_h.at[pl.ds(b0, NB)],
                                  osem1.at[b]).wait()
            pltpu.make_async_copy(trows.at[b], out2_h.at[pl.ds(b0, NB)],
                                  osem2.at[b]).wait()

        fire(0, 0)

        def superstep(s, carry):
            for bb in range(NBUF):
                i = s * NBUF + bb
                b0 = base_w + i * NB
                wait_gathers(bb)

                nb_slot = (bb + 1) % NBUF

                @pl.when(i + 1 < n_chunks)
                def _(i=i, nb_slot=nb_slot):
                    @pl.when(i >= NBUF - 1)
                    def _():
                        wait_out(i + 1 - NBUF, nb_slot)
                    fire(i + 1, nb_slot)

                for q in range(NB):
                    def add_row(l, c2, bb=bb, q=q):
                        for c in range(D // 16):
                            sl = pl.ds(c * 16, 16)
                            usum[bb, q, l, sl] = (upad[bb, q, l, sl]
                                                  + trows[bb, q, l, sl])
                        return c2

                    lax.fori_loop(0, L, add_row, 0, unroll=2)
                pltpu.async_copy(usum.at[bb], out1_h.at[pl.ds(b0, NB)],
                                 osem1.at[bb])
                pltpu.async_copy(trows.at[bb], out2_h.at[pl.ds(b0, NB)],
                                 osem2.at[bb])
            return carry

        lax.fori_loop(0, n_chunks // NBUF, superstep, 0)

        for k in range(NBUF):
            ci = n_chunks - NBUF + k
            wait_out(ci, ci % NBUF)

    return sc_call


V_PAD = 1000008
_sc_call = _make_sc_call(4096, V_PAD)


@jax.jit
def kernel(users, items, types, user_table, item_table, type_table):
    del items, item_table
    u = users.astype(jnp.int32)
    t = types.astype(jnp.int32)
    upad = jnp.pad(user_table, ((0, V_PAD - user_table.shape[0]),
                                (0, DP - D)))
    return _sc_call(u, t, upad, type_table)


# speedup vs baseline: 1.0291x; 1.0021x over previous
import functools

import jax
import jax.numpy as jnp
from jax import lax
from jax.experimental import pallas as pl
from jax.experimental.pallas import tpu as pltpu
from jax.experimental.pallas import tpu_sc as plsc

D = 64
DP = 128
L = 50
_info = plsc.get_sparse_core_info()
NC, NS = _info.num_cores, _info.num_subcores
NW = NC * NS

NB = 4
NBUF = 2
NIDX = 4


def _make_sc_call(b_total: int):
    rows_w = b_total // NW
    n_chunks = rows_w // NB
    mesh = plsc.VectorSubcoreMesh(core_axis_name="c", subcore_axis_name="s")

    @functools.partial(
        pl.kernel,
        out_type=(
            jax.ShapeDtypeStruct((b_total, L, D), jnp.float32),
            jax.ShapeDtypeStruct((b_total, L, D), jnp.float32),
        ),
        mesh=mesh,
        scratch_types=[
            pltpu.VMEM((NIDX, NB, L), jnp.int32),
            pltpu.VMEM((NIDX, NB, L), jnp.int32),
            pltpu.VMEM((NBUF, NB, L, DP), jnp.float32),
            pltpu.VMEM((NBUF, NB, L, D), jnp.float32),
            pltpu.VMEM((NBUF, NB, L, D), jnp.float32),
            pltpu.SemaphoreType.DMA((NIDX,)),
            pltpu.SemaphoreType.DMA((NBUF,)),
            pltpu.SemaphoreType.DMA((NBUF,)),
            pltpu.SemaphoreType.DMA((NBUF,)),
            pltpu.SemaphoreType.DMA((NBUF,)),
        ],
        compiler_params=pltpu.CompilerParams(use_tc_tiling_on_sc=False),
    )
    def sc_call(users_h, types_h, utab_h, ttab_h, out1_h, out2_h,
                idx_u, idx_t, upad, trows, usum,
                isem, gsem_u, gsem_t, osem1, osem2):
        wid = lax.axis_index("s") * NC + lax.axis_index("c")
        base_w = wid * rows_w

        def idx_pf(ci, si):
            b0 = base_w + ci * NB
            pltpu.async_copy(users_h.at[pl.ds(b0, NB)], idx_u.at[si],
                             isem.at[si])
            pltpu.async_copy(types_h.at[pl.ds(b0, NB)], idx_t.at[si],
                             isem.at[si])

        def wait_idx(ci, si):
            b0 = base_w + ci * NB
            pltpu.make_async_copy(users_h.at[pl.ds(b0, NB)], idx_u.at[si],
                                  isem.at[si]).wait()
            pltpu.make_async_copy(types_h.at[pl.ds(b0, NB)], idx_t.at[si],
                                  isem.at[si]).wait()

        def fire(b, si):
            for r in range(NB):
                pltpu.async_copy(utab_h.at[idx_u.at[si, r]],
                                 upad.at[b, r], gsem_u.at[b])
                pltpu.async_copy(ttab_h.at[idx_t.at[si, r]],
                                 trows.at[b, r], gsem_t.at[b])

        def wait_gathers(b, si):
            for r in range(NB):
                pltpu.make_async_copy(utab_h.at[idx_u.at[si, r]],
                                      upad.at[b, r], gsem_u.at[b]).wait()
                pltpu.make_async_copy(ttab_h.at[idx_t.at[si, r]],
                                      trows.at[b, r], gsem_t.at[b]).wait()

        def wait_out(ci, b):
            b0 = base_w + ci * NB
            pltpu.make_async_copy(usum.at[b], out1_h.at[pl.ds(b0, NB)],
                                  osem1.at[b]).wait()
            pltpu.make_async_copy(trows.at[b], out2_h.at[pl.ds(b0, NB)],
                                  osem2.at[b]).wait()

        idx_pf(0, 0)
        idx_pf(1, 1)
        wait_idx(0, 0)
        fire(0, 0)

        def superstep(s, carry):
            for bb in range(NIDX):
                i = s * NIDX + bb
                b = bb % NBUF
                b0 = base_w + i * NB
                wait_gathers(b, bb)

                j_slot = (bb + 1) % NBUF
                j_islot = (bb + 1) % NIDX
                k_islot = (bb + 2) % NIDX

                @pl.when(i + 1 < n_chunks)
                def _(i=i, j_slot=j_slot, j_islot=j_islot):
                    @pl.when(i + 1 >= NBUF)
                    def _():
                        wait_out(i + 1 - NBUF, j_slot)
                    wait_idx(i + 1, j_islot)
                    fire(j_slot, j_islot)

                @pl.when(i + 2 < n_chunks)
                def _(i=i, k_islot=k_islot):
                    idx_pf(i + 2, k_islot)

                for q in range(NB):
                    def add_row(l, c2, b=b, q=q):
                        for c in range(D // 16):
                            sl = pl.ds(c * 16, 16)
                            usum[b, q, l, sl] = (upad[b, q, l, sl]
                                                 + trows[b, q, l, sl])
                        return c2

                    lax.fori_loop(0, L, add_row, 0, unroll=5)
                pltpu.async_copy(usum.at[b], out1_h.at[pl.ds(b0, NB)],
                                 osem1.at[b])
                pltpu.async_copy(trows.at[b], out2_h.at[pl.ds(b0, NB)],
                                 osem2.at[b])
            return carry

        lax.fori_loop(0, n_chunks // NIDX, superstep, 0)

        for k in range(NBUF):
            ci = n_chunks - NBUF + k
            wait_out(ci, ci % NBUF)

    return sc_call


V_PAD = 1000008
_sc_call = _make_sc_call(4096)


@jax.jit
def kernel(users, items, types, user_table, item_table, type_table):
    del items, item_table
    u = users.astype(jnp.int32)
    t = types.astype(jnp.int32)
    upad = jnp.pad(user_table, ((0, V_PAD - user_table.shape[0]),
                                (0, DP - D)))
    return _sc_call(u, t, upad, type_table)
